# Initial kernel scaffold; baseline (speedup 1.0000x reference)
import jax
import jax.numpy as jnp
from jax.experimental import pallas as pl


def kernel(node_feats, edge_attrs, edge_feats, lengths, W_scalar, W_up, W1, b1, W2, b2, W3, W_out, edge_index):
    raise NotImplementedError("write your pallas kernel here")


# speedup vs baseline: 1.0525x; 1.0525x over previous
import functools

import jax
import jax.numpy as jnp
from jax.experimental import pallas as pl
from jax.experimental.pallas import tpu as pltpu

N = 10000
E = 320000
D = 128
RB = 8
AVG_NEIGH = 32.0

EB = 2000
NB = 2000


def _node_kernel(nf_ref, wsc_ref, w1a_ref, w1b_ref, wup_ref, a_ref, b_ref, up_ref):
    nf = nf_ref[...]
    ns = jnp.dot(nf, wsc_ref[...], preferred_element_type=jnp.float32)
    a_ref[...] = jnp.dot(ns, w1a_ref[...], preferred_element_type=jnp.float32)
    b_ref[...] = jnp.dot(ns, w1b_ref[...], preferred_element_type=jnp.float32)
    up_ref[...] = jnp.dot(nf, wup_ref[...], preferred_element_type=jnp.float32)


def _node_precompute(node_feats, W_scalar, W1a, W1b, W_up):
    grid = (N // NB,)
    blk = pl.BlockSpec((NB, D), lambda i: (i, 0))
    wblk = pl.BlockSpec((D, D), lambda i: (0, 0))
    return pl.pallas_call(
        _node_kernel,
        grid=grid,
        in_specs=[blk, wblk, wblk, wblk, wblk],
        out_specs=[blk, blk, blk],
        out_shape=[jax.ShapeDtypeStruct((N, D), jnp.float32)] * 3,
    )(node_feats, W_scalar, W1a, W1b, W_up)


def _mlp_kernel(gs_ref, gr_ref, ef_ref, sc_ref, u_ref,
                w1c_ref, w2_ref, b2_ref, w3_ref, o_ref):
    pre = gs_ref[...] + gr_ref[...]
    pre = pre + jnp.dot(ef_ref[...], w1c_ref[...], preferred_element_type=jnp.float32)
    h = pre * jax.nn.sigmoid(pre)
    pre2 = jnp.dot(h, w2_ref[...], preferred_element_type=jnp.float32) + b2_ref[...]
    h2 = pre2 * jax.nn.sigmoid(pre2)
    t = jnp.dot(h2, w3_ref[...], preferred_element_type=jnp.float32)
    o_ref[...] = u_ref[...] * sc_ref[...] * t


def _edge_mlp(gs, gr, ef_aug, scale, u, W1c_aug, W2, b2, W3):
    grid = (E // EB,)
    eblk = pl.BlockSpec((EB, D), lambda i: (i, 0))
    return pl.pallas_call(
        _mlp_kernel,
        grid=grid,
        in_specs=[
            eblk,
            eblk,
            pl.BlockSpec((EB, RB + 8), lambda i: (i, 0)),
            pl.BlockSpec((EB, 1), lambda i: (i, 0)),
            eblk,
            pl.BlockSpec((RB + 8, D), lambda i: (0, 0)),
            pl.BlockSpec((D, D), lambda i: (0, 0)),
            pl.BlockSpec((1, D), lambda i: (0, 0)),
            pl.BlockSpec((D, D), lambda i: (0, 0)),
        ],
        out_specs=eblk,
        out_shape=jax.ShapeDtypeStruct((E, D), jnp.float32),
    )(gs, gr, ef_aug, scale, u, W1c_aug, W2, b2, W3)


def _final_kernel(m_ref, wout_ref, o_ref):
    o_ref[...] = jnp.dot(m_ref[...], wout_ref[...],
                         preferred_element_type=jnp.float32) * (1.0 / AVG_NEIGH)


def _final(message, W_out):
    grid = (N // NB,)
    blk = pl.BlockSpec((NB, D), lambda i: (i, 0))
    return pl.pallas_call(
        _final_kernel,
        grid=grid,
        in_specs=[blk, pl.BlockSpec((D, D), lambda i: (0, 0))],
        out_specs=blk,
        out_shape=jax.ShapeDtypeStruct((N, D), jnp.float32),
    )(message, W_out)


def kernel(node_feats, edge_attrs, edge_feats, lengths, W_scalar, W_up,
           W1, b1, W2, b2, W3, W_out, edge_index):
    sender = edge_index[0]
    receiver = edge_index[1]
    W1a = W1[:D]
    W1b = W1[D:2 * D]
    W1c_aug = jnp.concatenate(
        [W1[2 * D:], b1[None, :], jnp.zeros((16 - RB - 2, D), jnp.float32)], axis=0)
    ef_aug = jnp.concatenate(
        [edge_feats, lengths, jnp.ones((E, 1), jnp.float32),
         jnp.zeros((E, 16 - RB - 2), jnp.float32)], axis=1)

    a, b, up = _node_precompute(node_feats, W_scalar, W1a, W1b, W_up)

    gs = jnp.take(a, sender, axis=0)
    gr = jnp.take(b, receiver, axis=0)
    u = jnp.take(up, sender, axis=0)

    mji = _edge_mlp(gs, gr, ef_aug, edge_attrs, u, W1c_aug, W2, b2[None, :], W3)

    message = jax.ops.segment_sum(mji, receiver, num_segments=N)

    out = _final(message, W_out)
    return out.reshape(N, D, 1)


# speedup vs baseline: 2.2209x; 2.1102x over previous
import functools

import jax
import jax.numpy as jnp
from jax import lax
from jax.experimental import pallas as pl
from jax.experimental.pallas import tpu as pltpu
from jax.experimental.pallas import tpu_sc as plsc

N = 10000
E = 320000
D = 128
RB = 8
AVG_NEIGH = 32.0

EB = 2000
NB = 2000

NC = 2
NS = 16
NW = NC * NS
CH = 128
NCHUNK = E // CH
_BASE = NCHUNK // NW
_EXTRA = NCHUNK - _BASE * NW


def _node_kernel(nf_ref, wsc_ref, w1a_ref, w1b_ref, wup_ref, au_ref, b_ref):
    nf = nf_ref[...]
    ns = jnp.dot(nf, wsc_ref[...], preferred_element_type=jnp.float32)
    au_ref[:, :D] = jnp.dot(ns, w1a_ref[...], preferred_element_type=jnp.float32)
    au_ref[:, D:] = jnp.dot(nf, wup_ref[...], preferred_element_type=jnp.float32)
    b_ref[...] = jnp.dot(ns, w1b_ref[...], preferred_element_type=jnp.float32)


def _node_precompute(node_feats, W_scalar, W1a, W1b, W_up):
    grid = (N // NB,)
    blk = pl.BlockSpec((NB, D), lambda i: (i, 0))
    wblk = pl.BlockSpec((D, D), lambda i: (0, 0))
    return pl.pallas_call(
        _node_kernel,
        grid=grid,
        in_specs=[blk, wblk, wblk, wblk, wblk],
        out_specs=[pl.BlockSpec((NB, 2 * D), lambda i: (i, 0)), blk],
        out_shape=[jax.ShapeDtypeStruct((N, 2 * D), jnp.float32),
                   jax.ShapeDtypeStruct((N, D), jnp.float32)],
    )(node_feats, W_scalar, W1a, W1b, W_up)


_sc_mesh = plsc.VectorSubcoreMesh(core_axis_name="c", subcore_axis_name="s")


@functools.partial(
    pl.kernel,
    out_type=[jax.ShapeDtypeStruct((E, 2 * D), jnp.float32),
              jax.ShapeDtypeStruct((E, D), jnp.float32)],
    mesh=_sc_mesh,
    scratch_types=[
        pltpu.VMEM((CH,), jnp.int32),
        pltpu.VMEM((CH,), jnp.int32),
        pltpu.VMEM((CH, 2 * D), jnp.float32),
        pltpu.VMEM((CH, D), jnp.float32),
        pltpu.SemaphoreType.DMA,
        pltpu.SemaphoreType.DMA,
    ],
)
def _sc_gather(au_hbm, b_hbm, ei_hbm, gsu_hbm, gr_hbm,
               idx_s, idx_r, rows_au, rows_b, sem1, sem2):
    wid = lax.axis_index("s") * NC + lax.axis_index("c")
    nch = _BASE + jnp.where(wid < _EXTRA, 1, 0)

    def body(j, carry):
        cid = wid + j * NW
        pltpu.sync_copy(ei_hbm.at[0, cid], idx_s)
        pltpu.sync_copy(ei_hbm.at[1, cid], idx_r)
        d1 = pltpu.async_copy(au_hbm.at[idx_s], rows_au, sem1)
        d2 = pltpu.async_copy(b_hbm.at[idx_r], rows_b, sem2)
        d1.wait()
        d2.wait()
        pltpu.sync_copy(rows_au, gsu_hbm.at[pl.ds(cid * CH, CH)])
        pltpu.sync_copy(rows_b, gr_hbm.at[pl.ds(cid * CH, CH)])
        return carry

    lax.fori_loop(0, nch, body, 0)


def _mlp_kernel(gsu_ref, gr_ref, ef_ref, sc_ref,
                w1c_ref, w2_ref, b2_ref, w3_ref, o_ref):
    pre = gsu_ref[:, :D] + gr_ref[...]
    pre = pre + jnp.dot(ef_ref[...], w1c_ref[...], preferred_element_type=jnp.float32)
    h = pre * jax.nn.sigmoid(pre)
    pre2 = jnp.dot(h, w2_ref[...], preferred_element_type=jnp.float32) + b2_ref[...]
    h2 = pre2 * jax.nn.sigmoid(pre2)
    t = jnp.dot(h2, w3_ref[...], preferred_element_type=jnp.float32)
    o_ref[...] = gsu_ref[:, D:] * sc_ref[...] * t


def _edge_mlp(gsu, gr, ef_aug, scale, W1c_aug, W2, b2, W3):
    grid = (E // EB,)
    eblk = pl.BlockSpec((EB, D), lambda i: (i, 0))
    return pl.pallas_call(
        _mlp_kernel,
        grid=grid,
        in_specs=[
            pl.BlockSpec((EB, 2 * D), lambda i: (i, 0)),
            eblk,
            pl.BlockSpec((EB, RB + 8), lambda i: (i, 0)),
            pl.BlockSpec((EB, 1), lambda i: (i, 0)),
            pl.BlockSpec((RB + 8, D), lambda i: (0, 0)),
            pl.BlockSpec((D, D), lambda i: (0, 0)),
            pl.BlockSpec((1, D), lambda i: (0, 0)),
            pl.BlockSpec((D, D), lambda i: (0, 0)),
        ],
        out_specs=eblk,
        out_shape=jax.ShapeDtypeStruct((E, D), jnp.float32),
    )(gsu, gr, ef_aug, scale, W1c_aug, W2, b2, W3)


def _final_kernel(m_ref, wout_ref, o_ref):
    o_ref[...] = jnp.dot(m_ref[...], wout_ref[...],
                         preferred_element_type=jnp.float32) * (1.0 / AVG_NEIGH)


def _final(message, W_out):
    grid = (N // NB,)
    blk = pl.BlockSpec((NB, D), lambda i: (i, 0))
    return pl.pallas_call(
        _final_kernel,
        grid=grid,
        in_specs=[blk, pl.BlockSpec((D, D), lambda i: (0, 0))],
        out_specs=blk,
        out_shape=jax.ShapeDtypeStruct((N, D), jnp.float32),
    )(message, W_out)


def kernel(node_feats, edge_attrs, edge_feats, lengths, W_scalar, W_up,
           W1, b1, W2, b2, W3, W_out, edge_index):
    sender = edge_index[0]
    receiver = edge_index[1]
    W1a = W1[:D]
    W1b = W1[D:2 * D]
    W1c_aug = jnp.concatenate(
        [W1[2 * D:], b1[None, :], jnp.zeros((16 - RB - 2, D), jnp.float32)], axis=0)
    ef_aug = jnp.concatenate(
        [edge_feats, lengths, jnp.ones((E, 1), jnp.float32),
         jnp.zeros((E, 16 - RB - 2), jnp.float32)], axis=1)

    au, b = _node_precompute(node_feats, W_scalar, W1a, W1b, W_up)

    ei3 = edge_index.reshape(2, NCHUNK, CH)
    gsu, gr = _sc_gather(au, b, ei3)

    mji = _edge_mlp(gsu, gr, ef_aug, edge_attrs, W1c_aug, W2, b2[None, :], W3)

    message = jax.ops.segment_sum(mji, receiver, num_segments=N)

    out = _final(message, W_out)
    return out.reshape(N, D, 1)


# speedup vs baseline: 3.6771x; 1.6557x over previous
import functools

import jax
import jax.numpy as jnp
from jax import lax
from jax.experimental import pallas as pl
from jax.experimental.pallas import tpu as pltpu
from jax.experimental.pallas import tpu_sc as plsc

N = 10000
E = 320000
D = 128
RB = 8
AVG_NEIGH = 32.0

EB = 2000
NB = 2000

NC = 2
NS = 16
NW = NC * NS
CH = 128
NCHUNK = E // CH
_BASE = NCHUNK // NW
_EXTRA = NCHUNK - _BASE * NW


def _node_kernel(nf_ref, wsc_ref, w1a_ref, w1b_ref, wup_ref, au_ref, b_ref):
    nf = nf_ref[...]
    ns = jnp.dot(nf, wsc_ref[...], preferred_element_type=jnp.float32)
    au_ref[:, :D] = jnp.dot(ns, w1a_ref[...], preferred_element_type=jnp.float32)
    au_ref[:, D:] = jnp.dot(nf, wup_ref[...], preferred_element_type=jnp.float32)
    b_ref[...] = jnp.dot(ns, w1b_ref[...], preferred_element_type=jnp.float32)


def _node_precompute(node_feats, W_scalar, W1a, W1b, W_up):
    grid = (N // NB,)
    blk = pl.BlockSpec((NB, D), lambda i: (i, 0))
    wblk = pl.BlockSpec((D, D), lambda i: (0, 0))
    return pl.pallas_call(
        _node_kernel,
        grid=grid,
        in_specs=[blk, wblk, wblk, wblk, wblk],
        out_specs=[pl.BlockSpec((NB, 2 * D), lambda i: (i, 0)), blk],
        out_shape=[jax.ShapeDtypeStruct((N, 2 * D), jnp.float32),
                   jax.ShapeDtypeStruct((N, D), jnp.float32)],
    )(node_feats, W_scalar, W1a, W1b, W_up)


_sc_mesh = plsc.VectorSubcoreMesh(core_axis_name="c", subcore_axis_name="s")


@functools.partial(
    pl.kernel,
    out_type=[jax.ShapeDtypeStruct((E, 2 * D), jnp.float32),
              jax.ShapeDtypeStruct((E, D), jnp.float32)],
    mesh=_sc_mesh,
    scratch_types=[
        pltpu.VMEM((CH,), jnp.int32),
        pltpu.VMEM((CH,), jnp.int32),
        pltpu.VMEM((CH, 2 * D), jnp.float32),
        pltpu.VMEM((CH, D), jnp.float32),
        pltpu.SemaphoreType.DMA,
        pltpu.SemaphoreType.DMA,
    ],
)
def _sc_gather(au_hbm, b_hbm, ei_hbm, gsu_hbm, gr_hbm,
               idx_s, idx_r, rows_au, rows_b, sem1, sem2):
    wid = lax.axis_index("s") * NC + lax.axis_index("c")
    nch = _BASE + jnp.where(wid < _EXTRA, 1, 0)

    def body(j, carry):
        cid = wid + j * NW
        pltpu.sync_copy(ei_hbm.at[0, cid], idx_s)
        pltpu.sync_copy(ei_hbm.at[1, cid], idx_r)
        d1 = pltpu.async_copy(au_hbm.at[idx_s], rows_au, sem1)
        d2 = pltpu.async_copy(b_hbm.at[idx_r], rows_b, sem2)
        d1.wait()
        d2.wait()
        pltpu.sync_copy(rows_au, gsu_hbm.at[pl.ds(cid * CH, CH)])
        pltpu.sync_copy(rows_b, gr_hbm.at[pl.ds(cid * CH, CH)])
        return carry

    lax.fori_loop(0, nch, body, 0)


N_PAD = 10240
_RPT = N_PAD // NS
_ZR = 160


@functools.partial(
    pl.kernel,
    out_type=jax.ShapeDtypeStruct((NC, N_PAD, D), jnp.float32),
    mesh=_sc_mesh,
    scratch_types=[
        pltpu.VMEM((CH,), jnp.int32),
        pltpu.VMEM((CH, D), jnp.float32),
        pltpu.VMEM((_ZR, D), jnp.float32),
        pltpu.VMEM_SHARED((N_PAD, D), jnp.float32),
    ],
)
def _sc_scatter(mji_hbm, ri_hbm, out_hbm, idx_r, rows, zbuf, acc):
    c = lax.axis_index("c")
    s = lax.axis_index("s")
    wid = s * NC + c
    nch = _BASE + jnp.where(wid < _EXTRA, 1, 0)

    def zrow(i, carry):
        for k in range(D // 16):
            zbuf[i, pl.ds(k * 16, 16)] = jnp.zeros((16,), jnp.float32)
        return carry

    lax.fori_loop(0, _ZR, zrow, 0)
    for t in range(_RPT // _ZR):
        pltpu.sync_copy(zbuf, acc.at[pl.ds(s * _RPT + t * _ZR, _ZR)])
    plsc.subcore_barrier()

    def body(j, carry):
        cid = wid + j * NW
        pltpu.sync_copy(ri_hbm.at[cid], idx_r)
        pltpu.sync_copy(mji_hbm.at[pl.ds(cid * CH, CH)], rows)
        pltpu.sync_copy(rows, acc.at[idx_r], add=True)
        return carry

    lax.fori_loop(0, nch, body, 0)
    plsc.subcore_barrier()
    for t in range(_RPT // _ZR):
        sl = pl.ds(s * _RPT + t * _ZR, _ZR)
        pltpu.sync_copy(acc.at[sl], out_hbm.at[c, sl])


def _mlp_kernel(gsu_ref, gr_ref, ef_ref, sc_ref,
                w1c_ref, w2_ref, b2_ref, w3_ref, o_ref):
    pre = gsu_ref[:, :D] + gr_ref[...]
    pre = pre + jnp.dot(ef_ref[...], w1c_ref[...], preferred_element_type=jnp.float32)
    h = pre * jax.nn.sigmoid(pre)
    pre2 = jnp.dot(h, w2_ref[...], preferred_element_type=jnp.float32) + b2_ref[...]
    h2 = pre2 * jax.nn.sigmoid(pre2)
    t = jnp.dot(h2, w3_ref[...], preferred_element_type=jnp.float32)
    o_ref[...] = gsu_ref[:, D:] * sc_ref[...] * t


def _edge_mlp(gsu, gr, ef_aug, scale, W1c_aug, W2, b2, W3):
    grid = (E // EB,)
    eblk = pl.BlockSpec((EB, D), lambda i: (i, 0))
    return pl.pallas_call(
        _mlp_kernel,
        grid=grid,
        in_specs=[
            pl.BlockSpec((EB, 2 * D), lambda i: (i, 0)),
            eblk,
            pl.BlockSpec((EB, RB + 8), lambda i: (i, 0)),
            pl.BlockSpec((EB, 1), lambda i: (i, 0)),
            pl.BlockSpec((RB + 8, D), lambda i: (0, 0)),
            pl.BlockSpec((D, D), lambda i: (0, 0)),
            pl.BlockSpec((1, D), lambda i: (0, 0)),
            pl.BlockSpec((D, D), lambda i: (0, 0)),
        ],
        out_specs=eblk,
        out_shape=jax.ShapeDtypeStruct((E, D), jnp.float32),
    )(gsu, gr, ef_aug, scale, W1c_aug, W2, b2, W3)


def _final_kernel(m_ref, wout_ref, o_ref):
    m = m_ref[0] + m_ref[1]
    o_ref[...] = jnp.dot(m, wout_ref[...],
                         preferred_element_type=jnp.float32) * (1.0 / AVG_NEIGH)


def _final(partials, W_out):
    grid = (N // NB,)
    blk = pl.BlockSpec((NB, D), lambda i: (i, 0))
    return pl.pallas_call(
        _final_kernel,
        grid=grid,
        in_specs=[pl.BlockSpec((NC, NB, D), lambda i: (0, i, 0)),
                  pl.BlockSpec((D, D), lambda i: (0, 0))],
        out_specs=blk,
        out_shape=jax.ShapeDtypeStruct((N, D), jnp.float32),
    )(partials, W_out)


def kernel(node_feats, edge_attrs, edge_feats, lengths, W_scalar, W_up,
           W1, b1, W2, b2, W3, W_out, edge_index):
    sender = edge_index[0]
    receiver = edge_index[1]
    W1a = W1[:D]
    W1b = W1[D:2 * D]
    W1c_aug = jnp.concatenate(
        [W1[2 * D:], b1[None, :], jnp.zeros((16 - RB - 2, D), jnp.float32)], axis=0)
    ef_aug = jnp.concatenate(
        [edge_feats, lengths, jnp.ones((E, 1), jnp.float32),
         jnp.zeros((E, 16 - RB - 2), jnp.float32)], axis=1)

    au, b = _node_precompute(node_feats, W_scalar, W1a, W1b, W_up)

    ei3 = edge_index.reshape(2, NCHUNK, CH)
    gsu, gr = _sc_gather(au, b, ei3)

    mji = _edge_mlp(gsu, gr, ef_aug, edge_attrs, W1c_aug, W2, b2[None, :], W3)

    partials = _sc_scatter(mji, receiver.reshape(NCHUNK, CH))

    out = _final(partials, W_out)
    return out.reshape(N, D, 1)


# speedup vs baseline: 4.2515x; 1.1562x over previous
import functools

import jax
import jax.numpy as jnp
from jax import lax
from jax.experimental import pallas as pl
from jax.experimental.pallas import tpu as pltpu
from jax.experimental.pallas import tpu_sc as plsc

N = 10000
E = 320000
D = 128
RB = 8
AVG_NEIGH = 32.0

EB = 2000
NB = 2000

NC = 2
NS = 16
NW = NC * NS
CH = 128
NCHUNK = E // CH
_BASE = NCHUNK // NW
_EXTRA = NCHUNK - _BASE * NW


def _node_kernel(nf_ref, wsc_ref, w1a_ref, w1b_ref, wup_ref, au_ref, b_ref):
    nf = nf_ref[...]
    ns = jnp.dot(nf, wsc_ref[...], preferred_element_type=jnp.float32)
    au_ref[:, :D] = jnp.dot(ns, w1a_ref[...], preferred_element_type=jnp.float32)
    au_ref[:, D:] = jnp.dot(nf, wup_ref[...], preferred_element_type=jnp.float32)
    b_ref[...] = jnp.dot(ns, w1b_ref[...], preferred_element_type=jnp.float32)


def _node_precompute(node_feats, W_scalar, W1a, W1b, W_up):
    grid = (N // NB,)
    blk = pl.BlockSpec((NB, D), lambda i: (i, 0))
    wblk = pl.BlockSpec((D, D), lambda i: (0, 0))
    return pl.pallas_call(
        _node_kernel,
        grid=grid,
        in_specs=[blk, wblk, wblk, wblk, wblk],
        out_specs=[pl.BlockSpec((NB, 2 * D), lambda i: (i, 0)), blk],
        out_shape=[jax.ShapeDtypeStruct((N, 2 * D), jnp.float32),
                   jax.ShapeDtypeStruct((N, D), jnp.float32)],
    )(node_feats, W_scalar, W1a, W1b, W_up)


_sc_mesh = plsc.VectorSubcoreMesh(core_axis_name="c", subcore_axis_name="s")


@functools.partial(
    pl.kernel,
    out_type=[jax.ShapeDtypeStruct((E, 2 * D), jnp.float32),
              jax.ShapeDtypeStruct((E, D), jnp.float32)],
    mesh=_sc_mesh,
    scratch_types=[
        pltpu.VMEM((2, CH), jnp.int32),
        pltpu.VMEM((2, CH), jnp.int32),
        pltpu.VMEM((CH, 2 * D), jnp.float32),
        pltpu.VMEM((CH, 2 * D), jnp.float32),
        pltpu.VMEM((CH, D), jnp.float32),
        pltpu.VMEM((CH, D), jnp.float32),
        pltpu.SemaphoreType.DMA,
        pltpu.SemaphoreType.DMA,
        pltpu.SemaphoreType.DMA,
        pltpu.SemaphoreType.DMA,
        pltpu.SemaphoreType.DMA,
        pltpu.SemaphoreType.DMA,
    ],
)
def _sc_gather(au_hbm, b_hbm, ei_hbm, gsu_hbm, gr_hbm,
               idx0, idx1, au0, au1, b0, b1, si0, si1, sg0, sg1, sw0, sw1):
    wid = lax.axis_index("s") * NC + lax.axis_index("c")

    def start_idx(j, idxb, sem):
        pltpu.async_copy(ei_hbm.at[wid + j * NW], idxb, sem)

    def wait_idx(idxb, sem):
        pltpu.make_async_copy(ei_hbm.at[0], idxb, sem).wait()

    def start_gather(idxb, aub, bb, sem):
        pltpu.async_copy(au_hbm.at[idxb.at[0]], aub, sem)
        pltpu.async_copy(b_hbm.at[idxb.at[1]], bb, sem)

    def wait_gather(idxb, aub, bb, sem):
        pltpu.make_async_copy(au_hbm.at[idxb.at[0]], aub, sem).wait()
        pltpu.make_async_copy(b_hbm.at[idxb.at[1]], bb, sem).wait()

    def start_write(j, aub, bb, sem):
        cid = wid + j * NW
        pltpu.async_copy(aub, gsu_hbm.at[pl.ds(cid * CH, CH)], sem)
        pltpu.async_copy(bb, gr_hbm.at[pl.ds(cid * CH, CH)], sem)

    def wait_write(aub, bb, sem):
        pltpu.make_async_copy(aub, gsu_hbm.at[pl.ds(0, CH)], sem).wait()
        pltpu.make_async_copy(bb, gr_hbm.at[pl.ds(0, CH)], sem).wait()

    start_idx(0, idx0, si0)
    start_idx(1, idx1, si1)

    def body(jj, carry):
        wait_idx(idx0, si0)

        @pl.when(jj > 0)
        def _():
            wait_write(au0, b0, sw0)

        start_gather(idx0, au0, b0, sg0)
        wait_idx(idx1, si1)

        @pl.when(jj > 0)
        def _():
            wait_write(au1, b1, sw1)

        start_gather(idx1, au1, b1, sg1)
        wait_gather(idx0, au0, b0, sg0)

        @pl.when(jj < _BASE // 2 - 1)
        def _():
            start_idx(2 * jj + 2, idx0, si0)

        start_write(2 * jj, au0, b0, sw0)
        wait_gather(idx1, au1, b1, sg1)

        @pl.when(jj < _BASE // 2 - 1)
        def _():
            start_idx(2 * jj + 3, idx1, si1)

        start_write(2 * jj + 1, au1, b1, sw1)
        return carry

    lax.fori_loop(0, _BASE // 2, body, 0)
    wait_write(au0, b0, sw0)
    wait_write(au1, b1, sw1)

    @pl.when(wid < _EXTRA)
    def _():
        j = _BASE
        start_idx(j, idx0, si0)
        wait_idx(idx0, si0)
        start_gather(idx0, au0, b0, sg0)
        wait_gather(idx0, au0, b0, sg0)
        start_write(j, au0, b0, sw0)
        wait_write(au0, b0, sw0)


N_PAD = 10240
_RPT = N_PAD // NS
_ZR = 32


@functools.partial(
    pl.kernel,
    out_type=jax.ShapeDtypeStruct((NC, N_PAD, D), jnp.float32),
    mesh=_sc_mesh,
    scratch_types=[
        pltpu.VMEM((CH,), jnp.int32),
        pltpu.VMEM((CH,), jnp.int32),
        pltpu.VMEM((CH, D), jnp.float32),
        pltpu.VMEM((CH, D), jnp.float32),
        pltpu.VMEM((_ZR, D), jnp.float32),
        pltpu.VMEM_SHARED((N_PAD, D), jnp.float32),
        pltpu.SemaphoreType.DMA,
        pltpu.SemaphoreType.DMA,
    ],
)
def _sc_scatter(mji_hbm, ri_hbm, out_hbm, idx0, idx1, rows0, rows1, zbuf,
                acc, sl0, sl1):
    c = lax.axis_index("c")
    s = lax.axis_index("s")
    wid = s * NC + c

    def zrow(i, carry):
        for k in range(D // 16):
            zbuf[i, pl.ds(k * 16, 16)] = jnp.zeros((16,), jnp.float32)
        return carry

    lax.fori_loop(0, _ZR, zrow, 0)
    for t in range(_RPT // _ZR):
        pltpu.sync_copy(zbuf, acc.at[pl.ds(s * _RPT + t * _ZR, _ZR)])
    plsc.subcore_barrier()

    def start_load(j, idxb, rowsb, sem):
        cid = wid + j * NW
        pltpu.async_copy(ri_hbm.at[cid], idxb, sem)
        pltpu.async_copy(mji_hbm.at[pl.ds(cid * CH, CH)], rowsb, sem)

    def wait_load(idxb, rowsb, sem):
        pltpu.make_async_copy(ri_hbm.at[0], idxb, sem).wait()
        pltpu.make_async_copy(mji_hbm.at[pl.ds(0, CH)], rowsb, sem).wait()

    start_load(0, idx0, rows0, sl0)
    start_load(1, idx1, rows1, sl1)

    def body(jj, carry):
        wait_load(idx0, rows0, sl0)
        pltpu.sync_copy(rows0, acc.at[idx0], add=True)

        @pl.when(jj < _BASE // 2 - 1)
        def _():
            start_load(2 * jj + 2, idx0, rows0, sl0)

        wait_load(idx1, rows1, sl1)
        pltpu.sync_copy(rows1, acc.at[idx1], add=True)

        @pl.when(jj < _BASE // 2 - 1)
        def _():
            start_load(2 * jj + 3, idx1, rows1, sl1)

        return carry

    lax.fori_loop(0, _BASE // 2, body, 0)

    @pl.when(wid < _EXTRA)
    def _():
        start_load(_BASE, idx0, rows0, sl0)
        wait_load(idx0, rows0, sl0)
        pltpu.sync_copy(rows0, acc.at[idx0], add=True)

    plsc.subcore_barrier()
    sl = pl.ds(s * _RPT, _RPT)
    pltpu.sync_copy(acc.at[sl], out_hbm.at[c, sl])


def _mlp_kernel(gsu_ref, gr_ref, ef_ref, sc_ref,
                w1c_ref, w2_ref, b2_ref, w3_ref, o_ref):
    pre = gsu_ref[:, :D] + gr_ref[...]
    pre = pre + jnp.dot(ef_ref[...], w1c_ref[...], preferred_element_type=jnp.float32)
    h = pre * jax.nn.sigmoid(pre)
    pre2 = jnp.dot(h, w2_ref[...], preferred_element_type=jnp.float32) + b2_ref[...]
    h2 = pre2 * jax.nn.sigmoid(pre2)
    t = jnp.dot(h2, w3_ref[...], preferred_element_type=jnp.float32)
    o_ref[...] = gsu_ref[:, D:] * sc_ref[...] * t


def _edge_mlp(gsu, gr, ef_aug, scale, W1c_aug, W2, b2, W3):
    grid = (E // EB,)
    eblk = pl.BlockSpec((EB, D), lambda i: (i, 0))
    return pl.pallas_call(
        _mlp_kernel,
        grid=grid,
        in_specs=[
            pl.BlockSpec((EB, 2 * D), lambda i: (i, 0)),
            eblk,
            pl.BlockSpec((EB, RB + 8), lambda i: (i, 0)),
            pl.BlockSpec((EB, 1), lambda i: (i, 0)),
            pl.BlockSpec((RB + 8, D), lambda i: (0, 0)),
            pl.BlockSpec((D, D), lambda i: (0, 0)),
            pl.BlockSpec((1, D), lambda i: (0, 0)),
            pl.BlockSpec((D, D), lambda i: (0, 0)),
        ],
        out_specs=eblk,
        out_shape=jax.ShapeDtypeStruct((E, D), jnp.float32),
    )(gsu, gr, ef_aug, scale, W1c_aug, W2, b2, W3)


def _final_kernel(m_ref, wout_ref, o_ref):
    m = m_ref[0] + m_ref[1]
    o_ref[...] = jnp.dot(m, wout_ref[...],
                         preferred_element_type=jnp.float32) * (1.0 / AVG_NEIGH)


def _final(partials, W_out):
    grid = (N // NB,)
    blk = pl.BlockSpec((NB, D), lambda i: (i, 0))
    return pl.pallas_call(
        _final_kernel,
        grid=grid,
        in_specs=[pl.BlockSpec((NC, NB, D), lambda i: (0, i, 0)),
                  pl.BlockSpec((D, D), lambda i: (0, 0))],
        out_specs=blk,
        out_shape=jax.ShapeDtypeStruct((N, D), jnp.float32),
    )(partials, W_out)


def kernel(node_feats, edge_attrs, edge_feats, lengths, W_scalar, W_up,
           W1, b1, W2, b2, W3, W_out, edge_index):
    sender = edge_index[0]
    receiver = edge_index[1]
    W1a = W1[:D]
    W1b = W1[D:2 * D]
    W1c_aug = jnp.concatenate(
        [W1[2 * D:], b1[None, :], jnp.zeros((16 - RB - 2, D), jnp.float32)], axis=0)
    ef_aug = jnp.concatenate(
        [edge_feats, lengths, jnp.ones((E, 1), jnp.float32),
         jnp.zeros((E, 16 - RB - 2), jnp.float32)], axis=1)

    au, b = _node_precompute(node_feats, W_scalar, W1a, W1b, W_up)

    ei3 = edge_index.reshape(2, NCHUNK, CH).transpose(1, 0, 2)
    gsu, gr = _sc_gather(au, b, ei3)

    mji = _edge_mlp(gsu, gr, ef_aug, edge_attrs, W1c_aug, W2, b2[None, :], W3)

    partials = _sc_scatter(mji, receiver.reshape(NCHUNK, CH))

    out = _final(partials, W_out)
    return out.reshape(N, D, 1)


# speedup vs baseline: 4.9532x; 1.1651x over previous
import functools

import jax
import jax.numpy as jnp
from jax import lax
from jax.experimental import pallas as pl
from jax.experimental.pallas import tpu as pltpu
from jax.experimental.pallas import tpu_sc as plsc

N = 10000
E = 320000
D = 128
RB = 8
AVG_NEIGH = 32.0

EB = 2000
NB = 2000

NC = 2
NS = 16
NW = NC * NS
CH = 128
NCHUNK = E // CH
_BASE = NCHUNK // NW
_EXTRA = NCHUNK - _BASE * NW


def _node_kernel(nf_ref, wsc_ref, w1a_ref, w1b_ref, wup_ref, au_ref, b_ref):
    nf = nf_ref[...]
    ns = jnp.dot(nf, wsc_ref[...], preferred_element_type=jnp.float32)
    a = jnp.dot(ns, w1a_ref[...], preferred_element_type=jnp.float32)
    u = jnp.dot(nf, wup_ref[...], preferred_element_type=jnp.float32)
    ai = lax.bitcast_convert_type(a.astype(jnp.bfloat16).astype(jnp.float32),
                                  jnp.int32)
    ui = lax.bitcast_convert_type(u.astype(jnp.bfloat16).astype(jnp.float32),
                                  jnp.int32)
    packed = ai | lax.shift_right_logical(ui, 16)
    au_ref[...] = lax.bitcast_convert_type(packed, jnp.float32)
    b_ref[...] = jnp.dot(ns, w1b_ref[...], preferred_element_type=jnp.float32)


def _node_precompute(node_feats, W_scalar, W1a, W1b, W_up):
    grid = (N // NB,)
    blk = pl.BlockSpec((NB, D), lambda i: (i, 0))
    wblk = pl.BlockSpec((D, D), lambda i: (0, 0))
    return pl.pallas_call(
        _node_kernel,
        grid=grid,
        in_specs=[blk, wblk, wblk, wblk, wblk],
        out_specs=[blk, blk],
        out_shape=[jax.ShapeDtypeStruct((N, D), jnp.float32),
                   jax.ShapeDtypeStruct((N, D), jnp.float32)],
    )(node_feats, W_scalar, W1a, W1b, W_up)


_sc_mesh = plsc.VectorSubcoreMesh(core_axis_name="c", subcore_axis_name="s")


@functools.partial(
    pl.kernel,
    out_type=[jax.ShapeDtypeStruct((E, D), jnp.float32),
              jax.ShapeDtypeStruct((E, D), jnp.float32)],
    mesh=_sc_mesh,
    scratch_types=[
        pltpu.VMEM((2, CH), jnp.int32),
        pltpu.VMEM((2, CH), jnp.int32),
        pltpu.VMEM((CH, D), jnp.float32),
        pltpu.VMEM((CH, D), jnp.float32),
        pltpu.VMEM((CH, D), jnp.float32),
        pltpu.VMEM((CH, D), jnp.float32),
        pltpu.SemaphoreType.DMA,
        pltpu.SemaphoreType.DMA,
        pltpu.SemaphoreType.DMA,
        pltpu.SemaphoreType.DMA,
        pltpu.SemaphoreType.DMA,
        pltpu.SemaphoreType.DMA,
    ],
)
def _sc_gather(au_hbm, b_hbm, ei_hbm, gsu_hbm, gr_hbm,
               idx0, idx1, au0, au1, b0, b1, si0, si1, sg0, sg1, sw0, sw1):
    wid = lax.axis_index("s") * NC + lax.axis_index("c")

    def start_idx(j, idxb, sem):
        pltpu.async_copy(ei_hbm.at[wid + j * NW], idxb, sem)

    def wait_idx(idxb, sem):
        pltpu.make_async_copy(ei_hbm.at[0], idxb, sem).wait()

    def start_gather(idxb, aub, bb, sem):
        pltpu.async_copy(au_hbm.at[idxb.at[0]], aub, sem)
        pltpu.async_copy(b_hbm.at[idxb.at[1]], bb, sem)

    def wait_gather(idxb, aub, bb, sem):
        pltpu.make_async_copy(au_hbm.at[idxb.at[0]], aub, sem).wait()
        pltpu.make_async_copy(b_hbm.at[idxb.at[1]], bb, sem).wait()

    def start_write(j, aub, bb, sem):
        cid = wid + j * NW
        pltpu.async_copy(aub, gsu_hbm.at[pl.ds(cid * CH, CH)], sem)
        pltpu.async_copy(bb, gr_hbm.at[pl.ds(cid * CH, CH)], sem)

    def wait_write(aub, bb, sem):
        pltpu.make_async_copy(aub, gsu_hbm.at[pl.ds(0, CH)], sem).wait()
        pltpu.make_async_copy(bb, gr_hbm.at[pl.ds(0, CH)], sem).wait()

    start_idx(0, idx0, si0)
    start_idx(1, idx1, si1)

    def body(jj, carry):
        wait_idx(idx0, si0)

        @pl.when(jj > 0)
        def _():
            wait_write(au0, b0, sw0)

        start_gather(idx0, au0, b0, sg0)
        wait_idx(idx1, si1)

        @pl.when(jj > 0)
        def _():
            wait_write(au1, b1, sw1)

        start_gather(idx1, au1, b1, sg1)
        wait_gather(idx0, au0, b0, sg0)

        @pl.when(jj < _BASE // 2 - 1)
        def _():
            start_idx(2 * jj + 2, idx0, si0)

        start_write(2 * jj, au0, b0, sw0)
        wait_gather(idx1, au1, b1, sg1)

        @pl.when(jj < _BASE // 2 - 1)
        def _():
            start_idx(2 * jj + 3, idx1, si1)

        start_write(2 * jj + 1, au1, b1, sw1)
        return carry

    lax.fori_loop(0, _BASE // 2, body, 0)
    wait_write(au0, b0, sw0)
    wait_write(au1, b1, sw1)

    @pl.when(wid < _EXTRA)
    def _():
        j = _BASE
        start_idx(j, idx0, si0)
        wait_idx(idx0, si0)
        start_gather(idx0, au0, b0, sg0)
        wait_gather(idx0, au0, b0, sg0)
        start_write(j, au0, b0, sw0)
        wait_write(au0, b0, sw0)


N_PAD = 10240
_RPT = N_PAD // NS
_ZR = 32


@functools.partial(
    pl.kernel,
    out_type=jax.ShapeDtypeStruct((NC, N_PAD, D), jnp.float32),
    mesh=_sc_mesh,
    scratch_types=[
        pltpu.VMEM((CH,), jnp.int32),
        pltpu.VMEM((CH,), jnp.int32),
        pltpu.VMEM((CH, D), jnp.float32),
        pltpu.VMEM((CH, D), jnp.float32),
        pltpu.VMEM((_ZR, D), jnp.float32),
        pltpu.VMEM_SHARED((N_PAD, D), jnp.float32),
        pltpu.SemaphoreType.DMA,
        pltpu.SemaphoreType.DMA,
    ],
)
def _sc_scatter(mji_hbm, ri_hbm, out_hbm, idx0, idx1, rows0, rows1, zbuf,
                acc, sl0, sl1):
    c = lax.axis_index("c")
    s = lax.axis_index("s")
    wid = s * NC + c

    def zrow(i, carry):
        for k in range(D // 16):
            zbuf[i, pl.ds(k * 16, 16)] = jnp.zeros((16,), jnp.float32)
        return carry

    lax.fori_loop(0, _ZR, zrow, 0)
    for t in range(_RPT // _ZR):
        pltpu.sync_copy(zbuf, acc.at[pl.ds(s * _RPT + t * _ZR, _ZR)])
    plsc.subcore_barrier()

    def start_load(j, idxb, rowsb, sem):
        cid = wid + j * NW
        pltpu.async_copy(ri_hbm.at[cid], idxb, sem)
        pltpu.async_copy(mji_hbm.at[pl.ds(cid * CH, CH)], rowsb, sem)

    def wait_load(idxb, rowsb, sem):
        pltpu.make_async_copy(ri_hbm.at[0], idxb, sem).wait()
        pltpu.make_async_copy(mji_hbm.at[pl.ds(0, CH)], rowsb, sem).wait()

    start_load(0, idx0, rows0, sl0)
    start_load(1, idx1, rows1, sl1)

    def body(jj, carry):
        wait_load(idx0, rows0, sl0)
        pltpu.sync_copy(rows0, acc.at[idx0], add=True)

        @pl.when(jj < _BASE // 2 - 1)
        def _():
            start_load(2 * jj + 2, idx0, rows0, sl0)

        wait_load(idx1, rows1, sl1)
        pltpu.sync_copy(rows1, acc.at[idx1], add=True)

        @pl.when(jj < _BASE // 2 - 1)
        def _():
            start_load(2 * jj + 3, idx1, rows1, sl1)

        return carry

    lax.fori_loop(0, _BASE // 2, body, 0)

    @pl.when(wid < _EXTRA)
    def _():
        start_load(_BASE, idx0, rows0, sl0)
        wait_load(idx0, rows0, sl0)
        pltpu.sync_copy(rows0, acc.at[idx0], add=True)

    plsc.subcore_barrier()
    sl = pl.ds(s * _RPT, _RPT)
    pltpu.sync_copy(acc.at[sl], out_hbm.at[c, sl])


def _mlp_kernel(gsu_ref, gr_ref, ef_ref, sc_ref,
                w1c_ref, w2_ref, b2_ref, w3_ref, o_ref):
    xi = lax.bitcast_convert_type(gsu_ref[...], jnp.int32)
    gs = lax.bitcast_convert_type(xi & jnp.int32(-65536), jnp.float32)
    u = lax.bitcast_convert_type(lax.shift_left(xi, 16), jnp.float32)
    pre = gs + gr_ref[...]
    pre = pre + jnp.dot(ef_ref[...], w1c_ref[...], preferred_element_type=jnp.float32)
    h = pre * jax.nn.sigmoid(pre)
    pre2 = jnp.dot(h, w2_ref[...], preferred_element_type=jnp.float32) + b2_ref[...]
    h2 = pre2 * jax.nn.sigmoid(pre2)
    t = jnp.dot(h2, w3_ref[...], preferred_element_type=jnp.float32)
    o_ref[...] = u * sc_ref[...] * t


def _edge_mlp(gsu, gr, ef_aug, scale, W1c_aug, W2, b2, W3):
    grid = (E // EB,)
    eblk = pl.BlockSpec((EB, D), lambda i: (i, 0))
    return pl.pallas_call(
        _mlp_kernel,
        grid=grid,
        in_specs=[
            eblk,
            eblk,
            pl.BlockSpec((EB, RB + 8), lambda i: (i, 0)),
            pl.BlockSpec((EB, 1), lambda i: (i, 0)),
            pl.BlockSpec((RB + 8, D), lambda i: (0, 0)),
            pl.BlockSpec((D, D), lambda i: (0, 0)),
            pl.BlockSpec((1, D), lambda i: (0, 0)),
            pl.BlockSpec((D, D), lambda i: (0, 0)),
        ],
        out_specs=eblk,
        out_shape=jax.ShapeDtypeStruct((E, D), jnp.float32),
    )(gsu, gr, ef_aug, scale, W1c_aug, W2, b2, W3)


def _final_kernel(m_ref, wout_ref, o_ref):
    m = m_ref[0] + m_ref[1]
    o_ref[...] = jnp.dot(m, wout_ref[...],
                         preferred_element_type=jnp.float32) * (1.0 / AVG_NEIGH)


def _final(partials, W_out):
    grid = (N // NB,)
    blk = pl.BlockSpec((NB, D), lambda i: (i, 0))
    return pl.pallas_call(
        _final_kernel,
        grid=grid,
        in_specs=[pl.BlockSpec((NC, NB, D), lambda i: (0, i, 0)),
                  pl.BlockSpec((D, D), lambda i: (0, 0))],
        out_specs=blk,
        out_shape=jax.ShapeDtypeStruct((N, D), jnp.float32),
    )(partials, W_out)


def kernel(node_feats, edge_attrs, edge_feats, lengths, W_scalar, W_up,
           W1, b1, W2, b2, W3, W_out, edge_index):
    sender = edge_index[0]
    receiver = edge_index[1]
    W1a = W1[:D]
    W1b = W1[D:2 * D]
    W1c_aug = jnp.concatenate(
        [W1[2 * D:], b1[None, :], jnp.zeros((16 - RB - 2, D), jnp.float32)], axis=0)
    ef_aug = jnp.concatenate(
        [edge_feats, lengths, jnp.ones((E, 1), jnp.float32),
         jnp.zeros((E, 16 - RB - 2), jnp.float32)], axis=1)

    au, b = _node_precompute(node_feats, W_scalar, W1a, W1b, W_up)

    ei3 = edge_index.reshape(2, NCHUNK, CH).transpose(1, 0, 2)
    gsu, gr = _sc_gather(au, b, ei3)

    mji = _edge_mlp(gsu, gr, ef_aug, edge_attrs, W1c_aug, W2, b2[None, :], W3)

    partials = _sc_scatter(mji, receiver.reshape(NCHUNK, CH))

    out = _final(partials, W_out)
    return out.reshape(N, D, 1)
